# baseline (speedup 1.0000x reference)
import functools

import jax
import jax.numpy as jnp
from jax import lax
from jax.experimental import pallas as pl
from jax.experimental.pallas import tpu as pltpu
from jax.experimental.pallas import tpu_sc as plsc

_VOCAB = 1000
_B = 1024
_L = 50
_NW = 32
_BPW = _B // _NW
_HALF = _L // 2
_NCHUNK = _BPW * 2
_SLAB = 64

_mesh = plsc.VectorSubcoreMesh(core_axis_name="c", subcore_axis_name="s")


@functools.partial(
    pl.kernel,
    mesh=_mesh,
    out_type=jax.ShapeDtypeStruct((_B, _L, _VOCAB), jnp.float32),
    scratch_types=[
        pltpu.VMEM((_NCHUNK, _HALF), jnp.int32),
        pltpu.VMEM((2, _HALF, _VOCAB), jnp.float32),
        pltpu.VMEM_SHARED((_VOCAB, _VOCAB), jnp.float32),
        pltpu.SemaphoreType.DMA,
        pltpu.SemaphoreType.DMA,
    ],
    compiler_params=pltpu.CompilerParams(use_tc_tiling_on_sc=False),
)
def _embed(idx_hbm, table_hbm, out_hbm, idx_v, rows_v, table_sh, sem0, sem1):
    cid = lax.axis_index("c")
    sid = lax.axis_index("s")
    wid = sid * 2 + cid
    b0 = wid * _BPW

    @pl.when(sid < 15)
    def _():
        pltpu.sync_copy(
            table_hbm.at[pl.ds(sid * _SLAB, _SLAB)],
            table_sh.at[pl.ds(sid * _SLAB, _SLAB)],
        )

    @pl.when(sid == 15)
    def _():
        pltpu.sync_copy(
            table_hbm.at[pl.ds(15 * _SLAB, _VOCAB - 15 * _SLAB)],
            table_sh.at[pl.ds(15 * _SLAB, _VOCAB - 15 * _SLAB)],
        )

    pltpu.sync_copy(idx_hbm.at[wid], idx_v)
    plsc.subcore_barrier()

    pltpu.async_copy(table_sh.at[idx_v.at[0]], rows_v.at[0], sem0)

    def body(p, carry):
        b = b0 + p
        g0 = p * 2
        pltpu.make_async_copy(table_sh.at[idx_v.at[g0]], rows_v.at[0], sem0).wait()
        pltpu.async_copy(table_sh.at[idx_v.at[g0 + 1]], rows_v.at[1], sem1)
        pltpu.sync_copy(rows_v.at[0], out_hbm.at[b, pl.ds(0, _HALF)])

        pltpu.make_async_copy(table_sh.at[idx_v.at[g0 + 1]], rows_v.at[1], sem1).wait()

        @pl.when(g0 + 2 < _NCHUNK)
        def _():
            pltpu.async_copy(table_sh.at[idx_v.at[g0 + 2]], rows_v.at[0], sem0)

        pltpu.sync_copy(rows_v.at[1], out_hbm.at[b, pl.ds(_HALF, _HALF)])
        return carry

    lax.fori_loop(0, _BPW, body, 0)


def kernel(idx, targets, token_embedding_table):
    del targets
    idx3 = idx.reshape(_NW, _NCHUNK, _HALF).astype(jnp.int32)
    return _embed(idx3, token_embedding_table)
